# Initial kernel scaffold; baseline (speedup 1.0000x reference)
import jax
import jax.numpy as jnp
from jax.experimental import pallas as pl


def kernel(x, edge_index, W1, b1, W2, b2):
    raise NotImplementedError("write your pallas kernel here")


# speedup vs baseline: 14.6922x; 14.6922x over previous
import functools

import jax
import jax.numpy as jnp
from jax import lax
from jax.experimental import pallas as pl
from jax.experimental.pallas import tpu as pltpu
from jax.experimental.pallas import tpu_sc as plsc

_N = 10000
_E = 320000
_D = 128
_DH = 256
_NC = 2
_NS = 16
_NW = _NC * _NS
_EPW = _E // _NW
_CH = 80
_NCHUNK = _EPW // _CH
_NPAD = 10240
_RPS = _NPAD // _NS
_RB = 1024
_NB = _NPAD // _RB

_mesh = plsc.VectorSubcoreMesh(
    core_axis_name="c", subcore_axis_name="s",
    num_cores=_NC, num_subcores=_NS)


@functools.partial(
    pl.kernel,
    out_type=jax.ShapeDtypeStruct((_NC, _NPAD), jnp.float32),
    mesh=_mesh,
    scratch_types=[
        pltpu.VMEM((_CH,), jnp.int32),
        pltpu.VMEM((_CH,), jnp.float32),
        pltpu.VMEM((_RPS,), jnp.float32),
        pltpu.VMEM_SHARED((_NPAD,), jnp.float32),
    ],
)
def _sc_degree(dst_hbm, deg_out, dst_v, ones_v, zbuf_v, deg_sh):
    c = lax.axis_index("c")
    s = lax.axis_index("s")
    wid = c * _NS + s

    def fill_z(i, carry):
        zbuf_v[pl.ds(i * 16, 16)] = jnp.zeros((16,), jnp.float32)
        return carry
    lax.fori_loop(0, _RPS // 16, fill_z, 0)

    def fill_o(i, carry):
        ones_v[pl.ds(i * 16, 16)] = jnp.ones((16,), jnp.float32)
        return carry
    lax.fori_loop(0, _CH // 16, fill_o, 0)

    pltpu.sync_copy(zbuf_v, deg_sh.at[pl.ds(s * _RPS, _RPS)])
    plsc.subcore_barrier()

    def step(i, carry):
        base = wid * _EPW + i * _CH
        pltpu.sync_copy(dst_hbm.at[pl.ds(base, _CH)], dst_v)
        pltpu.sync_copy(ones_v, deg_sh.at[dst_v], add=True)
        return carry
    lax.fori_loop(0, _NCHUNK, step, 0)

    plsc.subcore_barrier()
    pltpu.sync_copy(deg_sh.at[pl.ds(s * _RPS, _RPS)],
                    deg_out.at[c, pl.ds(s * _RPS, _RPS)])


@functools.partial(
    pl.kernel,
    out_type=jax.ShapeDtypeStruct((_NC, _NPAD, _D), jnp.float32),
    mesh=_mesh,
    scratch_types=[
        pltpu.VMEM((_CH,), jnp.int32),
        pltpu.VMEM((_CH,), jnp.int32),
        pltpu.VMEM((_CH, _D), jnp.float32),
        pltpu.VMEM_SHARED((_NPAD, _D), jnp.float32),
        pltpu.SemaphoreType.DMA,
    ],
)
def _sc_aggregate(table_hbm, src_hbm, dst_hbm, acc_out,
                  src_v, dst_v, rows_v, acc_sh, sem):
    c = lax.axis_index("c")
    s = lax.axis_index("s")
    wid = c * _NS + s

    def fill_zr(j, carry):
        def fill_zc(k, carry2):
            rows_v[j, pl.ds(k * 16, 16)] = jnp.zeros((16,), jnp.float32)
            return carry2
        lax.fori_loop(0, _D // 16, fill_zc, 0)
        return carry
    lax.fori_loop(0, _CH, fill_zr, 0)

    def zcp(k, carry):
        pltpu.sync_copy(rows_v, acc_sh.at[pl.ds(s * _RPS + k * _CH, _CH)])
        return carry
    lax.fori_loop(0, _RPS // _CH, zcp, 0)
    plsc.subcore_barrier()

    def step(i, carry):
        base = wid * _EPW + i * _CH
        pltpu.sync_copy(src_hbm.at[pl.ds(base, _CH)], src_v)
        pltpu.sync_copy(dst_hbm.at[pl.ds(base, _CH)], dst_v)
        pltpu.async_copy(table_hbm.at[src_v], rows_v, sem).wait()
        pltpu.sync_copy(rows_v, acc_sh.at[dst_v], add=True)
        return carry
    lax.fori_loop(0, _NCHUNK, step, 0)

    plsc.subcore_barrier()
    pltpu.sync_copy(acc_sh.at[pl.ds(s * _RPS, _RPS)],
                    acc_out.at[c, pl.ds(s * _RPS, _RPS)])


def _dinv_block(d0_ref, d1_ref):
    return lax.rsqrt(d0_ref[0, 0, :] + d1_ref[0, 0, :] + 1.0)[:, None]


def _prescale_body(x_ref, d0_ref, d1_ref, xs_ref):
    xs_ref[...] = x_ref[...] * _dinv_block(d0_ref, d1_ref)


def _mlp_body(a0_ref, a1_ref, xs_ref, d0_ref, d1_ref, w1_ref, b1_ref, w2_ref,
              ps2_ref):
    d = _dinv_block(d0_ref, d1_ref)
    g = (a0_ref[...] + a1_ref[...] + xs_ref[...]) * d
    h = jnp.maximum(
        jnp.dot(g, w1_ref[...], preferred_element_type=jnp.float32)
        + b1_ref[...], 0.0)
    ps2_ref[...] = jnp.dot(h, w2_ref[...],
                           preferred_element_type=jnp.float32) * d


def _final_body(a0_ref, a1_ref, ps2_ref, d0_ref, d1_ref, b2_ref, out_ref):
    d = _dinv_block(d0_ref, d1_ref)
    out_ref[...] = jnp.maximum(
        (a0_ref[...] + a1_ref[...] + ps2_ref[...]) * d + b2_ref[...], 0.0)


_row_spec = pl.BlockSpec((_RB, _D), lambda i: (i, 0))
_deg_spec = pl.BlockSpec((1, 1, _RB), lambda i: (i, 0, 0))


def _tc_prescale(x, d0, d1):
    return pl.pallas_call(
        _prescale_body,
        grid=(_NB,),
        in_specs=[_row_spec, _deg_spec, _deg_spec],
        out_specs=_row_spec,
        out_shape=jax.ShapeDtypeStruct((_N, _D), jnp.float32),
    )(x, d0, d1)


def _tc_mlp(a0, a1, xs, d0, d1, W1, b1, W2):
    return pl.pallas_call(
        _mlp_body,
        grid=(_NB,),
        in_specs=[
            _row_spec, _row_spec, _row_spec, _deg_spec, _deg_spec,
            pl.BlockSpec((_D, _DH), lambda i: (0, 0)),
            pl.BlockSpec((1, _DH), lambda i: (0, 0)),
            pl.BlockSpec((_DH, _D), lambda i: (0, 0)),
        ],
        out_specs=_row_spec,
        out_shape=jax.ShapeDtypeStruct((_N, _D), jnp.float32),
    )(a0, a1, xs, d0, d1, W1, b1, W2)


def _tc_final(a0, a1, ps2, d0, d1, b2):
    return pl.pallas_call(
        _final_body,
        grid=(_NB,),
        in_specs=[
            _row_spec, _row_spec, _row_spec, _deg_spec, _deg_spec,
            pl.BlockSpec((1, _D), lambda i: (0, 0)),
        ],
        out_specs=_row_spec,
        out_shape=jax.ShapeDtypeStruct((_N, _D), jnp.float32),
    )(a0, a1, ps2, d0, d1, b2)


def kernel(x, edge_index, W1, b1, W2, b2):
    src = edge_index[0].astype(jnp.int32)
    dst = edge_index[1].astype(jnp.int32)

    degp = _sc_degree(dst)
    d0 = degp[0].reshape(_NB, 1, _RB)
    d1 = degp[1].reshape(_NB, 1, _RB)

    xs = _tc_prescale(x, d0, d1)
    acc1 = _sc_aggregate(xs, src, dst)
    ps2 = _tc_mlp(acc1[0, :_N], acc1[1, :_N], xs, d0, d1, W1,
                  b1[None, :], W2)
    acc2 = _sc_aggregate(ps2, src, dst)
    return _tc_final(acc2[0, :_N], acc2[1, :_N], ps2, d0, d1, b2[None, :])
